# Initial kernel scaffold; baseline (speedup 1.0000x reference)
import jax
import jax.numpy as jnp
from jax.experimental import pallas as pl


def kernel(input_ids, weight):
    raise NotImplementedError("write your pallas kernel here")


# speedup vs baseline: 1.8325x; 1.8325x over previous
import functools

import jax
import jax.numpy as jnp
from jax import lax
from jax.experimental import pallas as pl
from jax.experimental.pallas import tpu as pltpu
from jax.experimental.pallas import tpu_sc as plsc

_NC, _NS = 2, 16
_NW = _NC * _NS


@functools.lru_cache(maxsize=None)
def _make_gather(B, D, C):
    b_per_w = B // _NW
    n_chunks = b_per_w // C
    mesh = plsc.VectorSubcoreMesh(
        core_axis_name="c", subcore_axis_name="s",
        num_cores=_NC, num_subcores=_NS,
    )

    @functools.partial(
        pl.kernel,
        out_type=jax.ShapeDtypeStruct((B, D), jnp.float32),
        mesh=mesh,
        scratch_types=[
            pltpu.VMEM((C,), jnp.int32),
            pltpu.VMEM((C, D), jnp.float32),
            pltpu.SemaphoreType.DMA,
        ],
        compiler_params=pltpu.CompilerParams(use_tc_tiling_on_sc=False),
    )
    def k(ids_hbm, table_hbm, out_hbm, idx_v, rows_v, sem):
        wid = lax.axis_index("s") * _NC + lax.axis_index("c")
        base = wid * b_per_w

        @pl.loop(0, n_chunks)
        def _chunk(i):
            off = base + i * C
            pltpu.sync_copy(ids_hbm.at[pl.ds(off, C)], idx_v)
            pltpu.async_copy(table_hbm.at[idx_v], rows_v, sem).wait()
            pltpu.sync_copy(rows_v, out_hbm.at[pl.ds(off, C)])

    return k


def kernel(input_ids, weight):
    bt, h = input_ids.shape
    v, d = weight.shape
    b = bt * h
    ids = input_ids.reshape(b).astype(jnp.int32)
    out = _make_gather(b, d, 800)(ids, weight)
    return out.reshape(bt, h, d)


# speedup vs baseline: 1.8695x; 1.0202x over previous
import functools

import jax
import jax.numpy as jnp
from jax import lax
from jax.experimental import pallas as pl
from jax.experimental.pallas import tpu as pltpu
from jax.experimental.pallas import tpu_sc as plsc

_NC, _NS = 2, 16
_NW = _NC * _NS


@functools.lru_cache(maxsize=None)
def _make_gather(B, D, C):
    b_per_w = B // _NW
    n_chunks = b_per_w // C
    mesh = plsc.VectorSubcoreMesh(
        core_axis_name="c", subcore_axis_name="s",
        num_cores=_NC, num_subcores=_NS,
    )

    @functools.partial(
        pl.kernel,
        out_type=jax.ShapeDtypeStruct((B, D), jnp.float32),
        mesh=mesh,
        scratch_types=[
            pltpu.VMEM((b_per_w,), jnp.int32),
            pltpu.VMEM((C, D), jnp.float32),
            pltpu.VMEM((C, D), jnp.float32),
            pltpu.SemaphoreType.DMA,
            pltpu.SemaphoreType.DMA,
            pltpu.SemaphoreType.DMA,
            pltpu.SemaphoreType.DMA,
        ],
        compiler_params=pltpu.CompilerParams(use_tc_tiling_on_sc=False),
    )
    def k(ids_hbm, table_hbm, out_hbm, idx_v, rows0, rows1, sg0, sg1, ss0, ss1):
        wid = lax.axis_index("s") * _NC + lax.axis_index("c")
        base = wid * b_per_w
        pltpu.sync_copy(ids_hbm.at[pl.ds(base, b_per_w)], idx_v)

        rows, sg, ss = [rows0, rows1], [sg0, sg1], [ss0, ss1]
        gathers = [None, None]
        stores = [None, None]
        for g in range(n_chunks):
            b = g % 2
            if stores[b] is not None:
                stores[b].wait()
            gathers[b] = pltpu.async_copy(
                table_hbm.at[idx_v.at[pl.ds(g * C, C)]], rows[b], sg[b])
            if g >= 1:
                gathers[1 - b].wait()
                stores[1 - b] = pltpu.async_copy(
                    rows[1 - b], out_hbm.at[pl.ds(base + (g - 1) * C, C)],
                    ss[1 - b])
        last = n_chunks - 1
        b = last % 2
        gathers[b].wait()
        stores[b] = pltpu.async_copy(
            rows[b], out_hbm.at[pl.ds(base + last * C, C)], ss[b])
        stores[1 - b].wait()
        stores[b].wait()

    return k


def kernel(input_ids, weight):
    bt, h = input_ids.shape
    v, d = weight.shape
    b = bt * h
    ids = input_ids.reshape(b).astype(jnp.int32)
    out = _make_gather(b, d, 800)(ids, weight)
    return out.reshape(bt, h, d)


# speedup vs baseline: 1.8720x; 1.0013x over previous
import functools

import jax
import jax.numpy as jnp
from jax import lax
from jax.experimental import pallas as pl
from jax.experimental.pallas import tpu as pltpu
from jax.experimental.pallas import tpu_sc as plsc

_NC, _NS = 2, 16
_NW = _NC * _NS


@functools.lru_cache(maxsize=None)
def _make_gather(B, D, C):
    b_per_w = B // _NW
    n_chunks = b_per_w // C
    mesh = plsc.VectorSubcoreMesh(
        core_axis_name="c", subcore_axis_name="s",
        num_cores=_NC, num_subcores=_NS,
    )

    @functools.partial(
        pl.kernel,
        out_type=jax.ShapeDtypeStruct((B, D), jnp.float32),
        mesh=mesh,
        scratch_types=[
            pltpu.VMEM((b_per_w,), jnp.int32),
            pltpu.VMEM((C, D), jnp.float32),
            pltpu.VMEM((C, D), jnp.float32),
            pltpu.VMEM((C, D), jnp.float32),
            pltpu.SemaphoreType.DMA,
            pltpu.SemaphoreType.DMA,
            pltpu.SemaphoreType.DMA,
            pltpu.SemaphoreType.DMA,
            pltpu.SemaphoreType.DMA,
            pltpu.SemaphoreType.DMA,
        ],
        compiler_params=pltpu.CompilerParams(use_tc_tiling_on_sc=False),
    )
    def k(ids_hbm, table_hbm, out_hbm, idx_v, rows0, rows1, rows2,
          sg0, sg1, sg2, ss0, ss1, ss2):
        wid = lax.axis_index("s") * _NC + lax.axis_index("c")
        base = wid * b_per_w
        pltpu.sync_copy(ids_hbm.at[pl.ds(base, b_per_w)], idx_v)

        rows, sg, ss = [rows0, rows1, rows2], [sg0, sg1, sg2], [ss0, ss1, ss2]
        gathers = [None] * 3
        stores = [None] * 3

        def gather_chunk(g):
            b = g % 3
            gathers[b] = pltpu.async_copy(
                table_hbm.at[idx_v.at[pl.ds(g * C, C)]], rows[b], sg[b])

        def store_chunk(g):
            b = g % 3
            stores[b] = pltpu.async_copy(
                rows[b], out_hbm.at[pl.ds(base + g * C, C)], ss[b])

        gather_chunk(0)
        gather_chunk(1)
        for g in range(n_chunks):
            b = g % 3
            if g + 2 < n_chunks:
                bn = (g + 2) % 3
                if stores[bn] is not None:
                    stores[bn].wait()
                gather_chunk(g + 2)
            gathers[b].wait()
            store_chunk(g)
        for b in range(3):
            if stores[b] is not None:
                stores[b].wait()

    return k


def kernel(input_ids, weight):
    bt, h = input_ids.shape
    v, d = weight.shape
    b = bt * h
    ids = input_ids.reshape(b).astype(jnp.int32)
    out = _make_gather(b, d, 512)(ids, weight)
    return out.reshape(bt, h, d)
